# Initial kernel scaffold; baseline (speedup 1.0000x reference)
import jax
import jax.numpy as jnp
from jax.experimental import pallas as pl


def kernel(x, mask):
    raise NotImplementedError("write your pallas kernel here")


# speedup vs baseline: 1.1007x; 1.1007x over previous
import jax
import jax.numpy as jnp
from jax.experimental import pallas as pl
from jax.experimental.pallas import tpu as pltpu

_ROWS = 32


def _body(mask_ref, out_ref):
    s = jax.nn.sigmoid(mask_ref[...])
    out_ref[...] = jnp.broadcast_to(s, out_ref.shape)


def kernel(x, mask):
    b, h, k = x.shape
    rows = b * h
    out = pl.pallas_call(
        _body,
        grid=(rows // _ROWS,),
        in_specs=[pl.BlockSpec((1, k), lambda i: (0, 0))],
        out_specs=pl.BlockSpec((_ROWS, k), lambda i: (i, 0)),
        out_shape=jax.ShapeDtypeStruct((rows, k), x.dtype),
        compiler_params=pltpu.CompilerParams(
            dimension_semantics=("arbitrary",)
        ),
    )(mask.reshape(1, k))
    return out.reshape(b, h, k)
